# Initial kernel scaffold; baseline (speedup 1.0000x reference)
import jax
import jax.numpy as jnp
from jax.experimental import pallas as pl


def kernel(x, edge_index, W1, att_src1, att_dst1, bias1, W2, att_src2, att_dst2, bias2):
    raise NotImplementedError("write your pallas kernel here")


# speedup vs baseline: 19.4787x; 19.4787x over previous
import functools

import jax
import jax.numpy as jnp
from jax import lax
from jax.experimental import pallas as pl
from jax.experimental.pallas import tpu as pltpu
from jax.experimental.pallas import tpu_sc as plsc

N = 10000
E = 160000
IN_CH = 256
HID = 8
HEADS = 8
OUT_CH = 256

NPAD = 10112
NTILES = 16
NCORES = 2
EPT = 10240
K = 128
NCHUNK = EPT // K
ROWS_PER_TILE = NPAD // NTILES
DRAIN = 624

F32 = jnp.float32
I32 = jnp.int32


def _sc_edge_pass(C, n_heads, srcp, dstp, ats, atd, hflat, minitflat, dinit):
  mesh = plsc.VectorSubcoreMesh(core_axis_name="c", subcore_axis_name="s")
  out_type = (
      jax.ShapeDtypeStruct((NCORES * N, C), F32),
      jax.ShapeDtypeStruct((NCORES * N, 16), F32),
  )
  scratch = [
      pltpu.VMEM_SHARED((NPAD, C), F32),
      pltpu.VMEM_SHARED((NPAD, 16), F32),
      pltpu.VMEM((K,), I32),
      pltpu.VMEM((K,), I32),
      pltpu.VMEM((K, 16), F32),
      pltpu.VMEM((K, 16), F32),
      pltpu.VMEM((K, 16), F32),
      pltpu.VMEM((K * 16,), F32),
      pltpu.VMEM((K, C), F32),
      pltpu.SemaphoreType.DMA,
      pltpu.SemaphoreType.DMA,
      pltpu.SemaphoreType.DMA,
  ]

  @functools.partial(pl.kernel, out_type=out_type, mesh=mesh,
                     scratch_types=scratch,
                     compiler_params=pltpu.CompilerParams(
                         needs_layout_passes=False,
                         use_tc_tiling_on_sc=False))
  def k(src_h, dst_h, ats_h, atd_h, h_h, minit_h, dinit_h,
        mout_h, dout_h,
        macc, dacc, sidx, didx, av, bv, wv, wflat, hv, sem1, sem2, sem3):
    cid = lax.axis_index("c")
    sid = lax.axis_index("s")
    lanes = lax.iota(I32, 16)

    r0 = sid * ROWS_PER_TILE
    pltpu.sync_copy(minit_h.at[pl.ds(cid * NPAD + r0, ROWS_PER_TILE), :],
                    macc.at[pl.ds(r0, ROWS_PER_TILE), :])
    pltpu.sync_copy(dinit_h.at[pl.ds(r0, ROWS_PER_TILE), :],
                    dacc.at[pl.ds(r0, ROWS_PER_TILE), :])
    plsc.subcore_barrier()

    ebase = sid * EPT

    def chunk(j, carry):
      base = ebase + j * K
      pltpu.sync_copy(src_h.at[pl.ds(base, K)], sidx)
      pltpu.sync_copy(dst_h.at[pl.ds(base, K)], didx)
      cp1 = pltpu.async_copy(ats_h.at[sidx], av, sem1)
      cp2 = pltpu.async_copy(atd_h.at[didx], bv, sem2)
      cp1.wait()
      cp2.wait()

      def adj(i, c2):
        sidx[pl.ds(i * 16, 16)] = sidx[pl.ds(i * 16, 16)] + cid * N
        return c2
      lax.fori_loop(0, K // 16, adj, 0)

      cp3 = pltpu.async_copy(h_h.at[sidx], hv, sem3)

      def edge_w(i, c2):
        z = av[i] + bv[i]
        z = jnp.where(z >= 0.0, z, 0.2 * z)
        w = jnp.exp(z)
        wv[i] = w
        wflat[pl.ds(i * 16, 16)] = w
        return c2
      lax.fori_loop(0, K, edge_w, 0)
      cp3.wait()

      def edge_m(i, c2):
        base16 = lanes * 0 + i * 16
        for v in range(C // 16):
          if n_heads == 1:
            col = base16
          else:
            col = base16 + (lanes >> 3) + (4 * cid + 2 * v)
          wb = plsc.load_gather(wflat, [col])
          hv[i, pl.ds(16 * v, 16)] = hv[i, pl.ds(16 * v, 16)] * wb
        return c2
      lax.fori_loop(0, K, edge_m, 0)

      pltpu.sync_copy(hv, macc.at[didx], add=True)
      pltpu.sync_copy(wv, dacc.at[didx], add=True)
      return carry

    lax.fori_loop(0, NCHUNK, chunk, 0)
    plsc.subcore_barrier()

    q0 = sid * DRAIN
    pltpu.sync_copy(macc.at[pl.ds(q0, DRAIN), :],
                    mout_h.at[pl.ds(cid * N + q0, DRAIN), :])
    pltpu.sync_copy(dacc.at[pl.ds(q0, DRAIN), :],
                    dout_h.at[pl.ds(cid * N + q0, DRAIN), :])

    @pl.when(sid == NTILES - 1)
    def _drain_tail():
      t0 = NTILES * DRAIN
      pltpu.sync_copy(macc.at[pl.ds(t0, N - t0), :],
                      mout_h.at[pl.ds(cid * N + t0, N - t0), :])
      pltpu.sync_copy(dacc.at[pl.ds(t0, N - t0), :],
                      dout_h.at[pl.ds(cid * N + t0, N - t0), :])

  return k(srcp, dstp, ats, atd, hflat, minitflat, dinit)


NB = 1000
GRID = N // NB


def _stage_a(x, W1, As1p, Ad1p, R0, R1):
  C = 4 * HID

  def body(x_ref, w1_ref, as_ref, ad_ref, r0_ref, r1_ref,
           h1s_ref, ats_ref, atd_ref, minit_ref, dinit_ref):
    h = jnp.dot(x_ref[...], w1_ref[...], preferred_element_type=F32)
    asr = jnp.dot(h, as_ref[...], preferred_element_type=F32)
    adr = jnp.dot(h, ad_ref[...], preferred_element_type=F32)
    z = asr + adr
    w = jnp.exp(jnp.where(z >= 0.0, z, 0.2 * z))
    ats_ref[...] = asr
    atd_ref[...] = adr
    dinit_ref[...] = w
    h1s_ref[0] = h[:, :C]
    h1s_ref[1] = h[:, C:]
    minit_ref[0] = jnp.dot(w, r0_ref[...], preferred_element_type=F32) * h[:, :C]
    minit_ref[1] = jnp.dot(w, r1_ref[...], preferred_element_type=F32) * h[:, C:]

  outs = pl.pallas_call(
      body,
      grid=(GRID,),
      in_specs=[
          pl.BlockSpec((NB, IN_CH), lambda i: (i, 0)),
          pl.BlockSpec((IN_CH, HEADS * HID), lambda i: (0, 0)),
          pl.BlockSpec((HEADS * HID, 16), lambda i: (0, 0)),
          pl.BlockSpec((HEADS * HID, 16), lambda i: (0, 0)),
          pl.BlockSpec((16, C), lambda i: (0, 0)),
          pl.BlockSpec((16, C), lambda i: (0, 0)),
      ],
      out_specs=[
          pl.BlockSpec((2, NB, C), lambda i: (0, i, 0)),
          pl.BlockSpec((NB, 16), lambda i: (i, 0)),
          pl.BlockSpec((NB, 16), lambda i: (i, 0)),
          pl.BlockSpec((2, NB, C), lambda i: (0, i, 0)),
          pl.BlockSpec((NB, 16), lambda i: (i, 0)),
      ],
      out_shape=[
          jax.ShapeDtypeStruct((2, N, C), F32),
          jax.ShapeDtypeStruct((N, 16), F32),
          jax.ShapeDtypeStruct((N, 16), F32),
          jax.ShapeDtypeStruct((2, N, C), F32),
          jax.ShapeDtypeStruct((N, 16), F32),
      ],
  )(x, W1, As1p, Ad1p, R0, R1)
  return outs


def _stage_b(macc1, dacc1, W2, a2sp, a2dp, bias1, R0, R1):
  C1 = 4 * HID
  C2 = OUT_CH // 2

  def body(m_ref, d_ref, w2_ref, a2s_ref, a2d_ref, b1_ref, r0_ref, r1_ref,
           h2s_ref, ats_ref, atd_ref, minit_ref, dinit_ref):
    o0 = m_ref[0] / (jnp.dot(d_ref[0], r0_ref[...],
                             preferred_element_type=F32) + 1e-16)
    o1 = m_ref[1] / (jnp.dot(d_ref[1], r1_ref[...],
                             preferred_element_type=F32) + 1e-16)
    z = jnp.concatenate([o0, o1], axis=1) + b1_ref[...]
    z = jnp.where(z > 0.0, z, jnp.exp(z) - 1.0)
    h2 = jnp.dot(z, w2_ref[...], preferred_element_type=F32)
    asr = jnp.dot(h2, a2s_ref[...], preferred_element_type=F32)
    adr = jnp.dot(h2, a2d_ref[...], preferred_element_type=F32)
    zz = asr + adr
    w = jnp.exp(jnp.where(zz >= 0.0, zz, 0.2 * zz))
    ats_ref[...] = asr
    atd_ref[...] = adr
    dinit_ref[...] = w
    h2s_ref[0] = h2[:, :C2]
    h2s_ref[1] = h2[:, C2:]
    minit_ref[0] = h2[:, :C2] * w[:, 0:1]
    minit_ref[1] = h2[:, C2:] * w[:, 0:1]

  outs = pl.pallas_call(
      body,
      grid=(GRID,),
      in_specs=[
          pl.BlockSpec((2, NB, C1), lambda i: (0, i, 0)),
          pl.BlockSpec((2, NB, 16), lambda i: (0, i, 0)),
          pl.BlockSpec((HEADS * HID, OUT_CH), lambda i: (0, 0)),
          pl.BlockSpec((OUT_CH, 16), lambda i: (0, 0)),
          pl.BlockSpec((OUT_CH, 16), lambda i: (0, 0)),
          pl.BlockSpec((1, HEADS * HID), lambda i: (0, 0)),
          pl.BlockSpec((16, C1), lambda i: (0, 0)),
          pl.BlockSpec((16, C1), lambda i: (0, 0)),
      ],
      out_specs=[
          pl.BlockSpec((2, NB, C2), lambda i: (0, i, 0)),
          pl.BlockSpec((NB, 16), lambda i: (i, 0)),
          pl.BlockSpec((NB, 16), lambda i: (i, 0)),
          pl.BlockSpec((2, NB, C2), lambda i: (0, i, 0)),
          pl.BlockSpec((NB, 16), lambda i: (i, 0)),
      ],
      out_shape=[
          jax.ShapeDtypeStruct((2, N, C2), F32),
          jax.ShapeDtypeStruct((N, 16), F32),
          jax.ShapeDtypeStruct((N, 16), F32),
          jax.ShapeDtypeStruct((2, N, C2), F32),
          jax.ShapeDtypeStruct((N, 16), F32),
      ],
  )(macc1, dacc1, W2, a2sp, a2dp, bias1, R0, R1)
  return outs


def _stage_c(macc2, dacc2, bias2):
  C2 = OUT_CH // 2

  def body(m_ref, d_ref, b2_ref, out_ref):
    o0 = m_ref[0] / (d_ref[0][:, 0:1] + 1e-16)
    o1 = m_ref[1] / (d_ref[1][:, 0:1] + 1e-16)
    o = jnp.concatenate([o0, o1], axis=1) + b2_ref[...]
    m = jnp.max(o, axis=1, keepdims=True)
    s = o - m
    out_ref[...] = s - jnp.log(jnp.sum(jnp.exp(s), axis=1, keepdims=True))

  return pl.pallas_call(
      body,
      grid=(GRID,),
      in_specs=[
          pl.BlockSpec((2, NB, C2), lambda i: (0, i, 0)),
          pl.BlockSpec((2, NB, 16), lambda i: (0, i, 0)),
          pl.BlockSpec((1, OUT_CH), lambda i: (0, 0)),
      ],
      out_specs=pl.BlockSpec((NB, OUT_CH), lambda i: (i, 0)),
      out_shape=jax.ShapeDtypeStruct((N, OUT_CH), F32),
  )(macc2, dacc2, bias2)


def _pad_rows(a, rows):
  return jnp.concatenate(
      [a, jnp.zeros((rows - a.shape[0],) + a.shape[1:], a.dtype)], axis=0)


@jax.jit
def kernel(x, edge_index, W1, att_src1, att_dst1, bias1,
           W2, att_src2, att_dst2, bias2):
  src = edge_index[0].astype(I32).reshape(NTILES, E // NTILES)
  dst = edge_index[1].astype(I32).reshape(NTILES, E // NTILES)
  padn = EPT - E // NTILES
  srcp = jnp.concatenate(
      [src, jnp.zeros((NTILES, padn), I32)], axis=1).reshape(-1)
  dstp = jnp.concatenate(
      [dst, jnp.full((NTILES, padn), N, I32)], axis=1).reshape(-1)

  eye8 = jnp.eye(HEADS, dtype=F32)
  As1 = (att_src1[:, :, None] * eye8[:, None, :]).reshape(HEADS * HID, HEADS)
  Ad1 = (att_dst1[:, :, None] * eye8[:, None, :]).reshape(HEADS * HID, HEADS)
  zpad = jnp.zeros((HEADS * HID, 16 - HEADS), F32)
  As1p = jnp.concatenate([As1, zpad], axis=1)
  Ad1p = jnp.concatenate([Ad1, zpad], axis=1)

  g = jnp.arange(16)[:, None]
  ch = jnp.arange(4 * HID)[None, :]
  R0 = (g == ch // HID).astype(F32)
  R1 = (g == 4 + ch // HID).astype(F32)

  a2sp = jnp.concatenate(
      [att_src2.reshape(OUT_CH, 1), jnp.zeros((OUT_CH, 15), F32)], axis=1)
  a2dp = jnp.concatenate(
      [att_dst2.reshape(OUT_CH, 1), jnp.zeros((OUT_CH, 15), F32)], axis=1)

  h1s, ats1, atd1, minit1, dinit1 = _stage_a(x, W1, As1p, Ad1p, R0, R1)
  macc1, dacc1 = _sc_edge_pass(
      4 * HID, HEADS, srcp, dstp,
      _pad_rows(ats1, NPAD), _pad_rows(atd1, NPAD),
      h1s.reshape(NCORES * N, 4 * HID),
      jnp.concatenate([_pad_rows(minit1[0], NPAD),
                       _pad_rows(minit1[1], NPAD)], axis=0),
      _pad_rows(dinit1, NPAD))
  macc1 = macc1.reshape(NCORES, N, 4 * HID)
  dacc1 = dacc1.reshape(NCORES, N, 16)

  h2s, ats2, atd2, minit2, dinit2 = _stage_b(
      macc1, dacc1, W2, a2sp, a2dp, bias1.reshape(1, HEADS * HID), R0, R1)
  macc2, dacc2 = _sc_edge_pass(
      OUT_CH // 2, 1, srcp, dstp,
      _pad_rows(ats2, NPAD), _pad_rows(atd2, NPAD),
      h2s.reshape(NCORES * N, OUT_CH // 2),
      jnp.concatenate([_pad_rows(minit2[0], NPAD),
                       _pad_rows(minit2[1], NPAD)], axis=0),
      _pad_rows(dinit2, NPAD))
  macc2 = macc2.reshape(NCORES, N, OUT_CH // 2)
  dacc2 = dacc2.reshape(NCORES, N, 16)

  return _stage_c(macc2, dacc2, bias2.reshape(1, OUT_CH))


# speedup vs baseline: 32.5822x; 1.6727x over previous
import functools

import jax
import jax.numpy as jnp
from jax import lax
from jax.experimental import pallas as pl
from jax.experimental.pallas import tpu as pltpu
from jax.experimental.pallas import tpu_sc as plsc

N = 10000
E = 160000
IN_CH = 256
HID = 8
HEADS = 8
OUT_CH = 256

NPAD = 10008
NTILES = 16
NCORES = 2
EPT = 10240
INIT_RPT = 624
DRAIN = 624

F32 = jnp.float32
I32 = jnp.int32


def _sc_edge_pass(C, n_heads, K, srcp2, dstp, atd, htab, minit):
  CW = C + 16
  NCHUNK = EPT // K
  mesh = plsc.VectorSubcoreMesh(core_axis_name="c", subcore_axis_name="s")
  out_type = jax.ShapeDtypeStruct((NCORES * N, CW), F32)
  scratch = [
      pltpu.VMEM_SHARED((NPAD, CW), F32),
      [pltpu.VMEM((K,), I32) for _ in range(4)],
      [pltpu.VMEM((K,), I32) for _ in range(4)],
      [pltpu.VMEM((K, 16), F32) for _ in range(2)],
      [pltpu.VMEM((K, CW), F32) for _ in range(3)],
      [pltpu.SemaphoreType.DMA for _ in range(4)],
      [pltpu.SemaphoreType.DMA for _ in range(3)],
      [pltpu.SemaphoreType.DMA for _ in range(3)],
  ]

  @functools.partial(pl.kernel, out_type=out_type, mesh=mesh,
                     scratch_types=scratch,
                     compiler_params=pltpu.CompilerParams(
                         needs_layout_passes=False,
                         use_tc_tiling_on_sc=False))
  def k(src_h, dst_h, atd_h, htab_h, minit_h, mout_h,
        macc, sidx, didx, bv, hv, isem, gsem, ssem):
    cid = lax.axis_index("c")
    sid = lax.axis_index("s")
    lanes = lax.iota(I32, 16)

    r0 = sid * INIT_RPT
    pltpu.sync_copy(minit_h.at[pl.ds(cid * NPAD + r0, INIT_RPT), :],
                    macc.at[pl.ds(r0, INIT_RPT), :])

    @pl.when(sid == NTILES - 1)
    def _init_tail():
      t0 = NTILES * INIT_RPT
      pltpu.sync_copy(minit_h.at[pl.ds(cid * NPAD + t0, NPAD - t0), :],
                      macc.at[pl.ds(t0, NPAD - t0), :])

    plsc.subcore_barrier()

    def issue_i(j, s4):
      pltpu.async_copy(src_h.at[cid, sid, j], sidx[s4], isem[s4])
      pltpu.async_copy(dst_h.at[sid, j], didx[s4], isem[s4])

    def wait_i(j, s4):
      pltpu.make_async_copy(src_h.at[cid, sid, j], sidx[s4],
                            isem[s4]).wait()
      pltpu.make_async_copy(dst_h.at[sid, j], didx[s4], isem[s4]).wait()

    def issue_g(s4, b2, p):
      pltpu.async_copy(atd_h.at[didx[s4]], bv[b2], gsem[p])
      pltpu.async_copy(htab_h.at[sidx[s4]], hv[p], gsem[p])

    def wait_g(s4, b2, p):
      pltpu.make_async_copy(atd_h.at[didx[s4]], bv[b2], gsem[p]).wait()
      pltpu.make_async_copy(htab_h.at[sidx[s4]], hv[p], gsem[p]).wait()

    def issue_s(s4, p):
      pltpu.async_copy(hv[p], macc.at[didx[s4]], ssem[p], add=True)

    def wait_s(s4, p):
      pltpu.make_async_copy(hv[p], macc.at[didx[s4]], ssem[p]).wait()

    def compute(b2, p):
      hvp = hv[p]
      bvp = bv[b2]

      def edge(i):
        z = hvp[i, pl.ds(C, 16)] + bvp[i]
        z = jnp.where(z >= 0.0, z, 0.2 * z)
        w = jnp.exp(z)
        if n_heads == 1:
          wb = jnp.take_along_axis(w, lanes * 0, axis=0,
                                   mode="promise_in_bounds")
          for v in range(CW // 16):
            hvp[i, pl.ds(16 * v, 16)] = hvp[i, pl.ds(16 * v, 16)] * wb
        else:
          for v in range(C // 16):
            col = (lanes >> 3) + (4 * cid + 2 * v)
            wb = jnp.take_along_axis(w, col, axis=0,
                                     mode="promise_in_bounds")
            hvp[i, pl.ds(16 * v, 16)] = hvp[i, pl.ds(16 * v, 16)] * wb
          wb = jnp.take_along_axis(w, lanes & 7, axis=0,
                                   mode="promise_in_bounds")
          hvp[i, pl.ds(C, 16)] = hvp[i, pl.ds(C, 16)] * wb

      def body2(t, c2):
        edge(2 * t)
        edge(2 * t + 1)
        return c2
      lax.fori_loop(0, K // 2, body2, 0)

    def step(j, s4, s4n, s4n2, b2, b2n, p, q, first):
      wait_g(s4, b2, p)
      if not first:
        wait_s(s4n2, q)
      wait_i(j + 1, s4n)
      issue_g(s4n, b2n, q)
      issue_i(j + 2, s4n2)
      compute(b2, p)
      issue_s(s4, p)

    issue_i(0, 0)
    issue_i(1, 1)
    wait_i(0, 0)
    issue_g(0, 0, 0)
    step(0, 0, 1, 2, 0, 1, 0, 1, True)
    step(1, 1, 2, 3, 1, 0, 1, 2, True)

    M = (NCHUNK - 4) // 12

    def pipe12(t, c2):
      for b in range(12):
        j = 2 + 12 * t + b
        step(j, (2 + b) % 4, (3 + b) % 4, b % 4, b % 2, (1 + b) % 2,
             (2 + b) % 3, b % 3, False)
      return c2
    lax.fori_loop(0, M, pipe12, 0)

    for j in range(2 + 12 * M, NCHUNK):
      p = j % 3
      q = (j + 1) % 3
      wait_g(j % 4, j % 2, p)
      wait_s((j + 2) % 4, q)
      if j + 1 < NCHUNK:
        wait_i(j + 1, (j + 1) % 4)
        issue_g((j + 1) % 4, (j + 1) % 2, q)
      if j + 2 < NCHUNK:
        issue_i(j + 2, (j + 2) % 4)
      compute(j % 2, p)
      issue_s(j % 4, p)
    wait_s((NCHUNK - 2) % 4, (NCHUNK - 2) % 3)
    wait_s((NCHUNK - 1) % 4, (NCHUNK - 1) % 3)
    plsc.subcore_barrier()

    q0 = sid * DRAIN
    pltpu.sync_copy(macc.at[pl.ds(q0, DRAIN), :],
                    mout_h.at[pl.ds(cid * N + q0, DRAIN), :])

    @pl.when(sid == NTILES - 1)
    def _drain_tail():
      t0 = NTILES * DRAIN
      pltpu.sync_copy(macc.at[pl.ds(t0, N - t0), :],
                      mout_h.at[pl.ds(cid * N + t0, N - t0), :])

  return k(srcp2, dstp, atd, htab, minit)


NB = 1000
GRID = N // NB


def _stage_a(x, W1, As1p, Ad1p, R0, R1):
  C = 4 * HID
  CW = C + 16

  def body(x_ref, w1_ref, as_ref, ad_ref, r0_ref, r1_ref,
           htab_ref, atd_ref, minit_ref):
    h = jnp.dot(x_ref[...], w1_ref[...], preferred_element_type=F32)
    asr = jnp.dot(h, as_ref[...], preferred_element_type=F32)
    adr = jnp.dot(h, ad_ref[...], preferred_element_type=F32)
    z = asr + adr
    w = jnp.exp(jnp.where(z >= 0.0, z, 0.2 * z))
    atd_ref[...] = adr
    ones8 = jnp.ones((NB, 8), F32)
    zeros8 = jnp.zeros((NB, 8), F32)
    htab_ref[0] = jnp.concatenate([h[:, :C], asr[:, :8], ones8], axis=1)
    htab_ref[1] = jnp.concatenate([h[:, C:], asr[:, :8], ones8], axis=1)
    w0 = jnp.dot(w, r0_ref[...], preferred_element_type=F32)
    w1 = jnp.dot(w, r1_ref[...], preferred_element_type=F32)
    minit_ref[0] = jnp.concatenate([w0 * h[:, :C], zeros8, w[:, :8]], axis=1)
    minit_ref[1] = jnp.concatenate([w1 * h[:, C:], zeros8, w[:, :8]], axis=1)

  return pl.pallas_call(
      body,
      grid=(GRID,),
      in_specs=[
          pl.BlockSpec((NB, IN_CH), lambda i: (i, 0)),
          pl.BlockSpec((IN_CH, HEADS * HID), lambda i: (0, 0)),
          pl.BlockSpec((HEADS * HID, 16), lambda i: (0, 0)),
          pl.BlockSpec((HEADS * HID, 16), lambda i: (0, 0)),
          pl.BlockSpec((16, C), lambda i: (0, 0)),
          pl.BlockSpec((16, C), lambda i: (0, 0)),
      ],
      out_specs=[
          pl.BlockSpec((2, NB, CW), lambda i: (0, i, 0)),
          pl.BlockSpec((NB, 16), lambda i: (i, 0)),
          pl.BlockSpec((2, NB, CW), lambda i: (0, i, 0)),
      ],
      out_shape=[
          jax.ShapeDtypeStruct((2, N, CW), F32),
          jax.ShapeDtypeStruct((N, 16), F32),
          jax.ShapeDtypeStruct((2, N, CW), F32),
      ],
  )(x, W1, As1p, Ad1p, R0, R1)


def _stage_b(macc1, W2, a2sp, a2dp, bias1, D0, D1):
  C1 = 4 * HID
  CW1 = C1 + 16
  C2 = OUT_CH // 2
  CW2 = C2 + 16

  def body(m_ref, w2_ref, a2s_ref, a2d_ref, b1_ref, d0_ref, d1_ref,
           htab_ref, atd_ref, minit_ref):
    o0 = m_ref[0][:, :C1] / (jnp.dot(m_ref[0], d0_ref[...],
                                     preferred_element_type=F32) + 1e-16)
    o1 = m_ref[1][:, :C1] / (jnp.dot(m_ref[1], d1_ref[...],
                                     preferred_element_type=F32) + 1e-16)
    z = jnp.concatenate([o0, o1], axis=1) + b1_ref[...]
    z = jnp.where(z > 0.0, z, jnp.exp(z) - 1.0)
    h2 = jnp.dot(z, w2_ref[...], preferred_element_type=F32)
    asr = jnp.dot(h2, a2s_ref[...], preferred_element_type=F32)
    adr = jnp.dot(h2, a2d_ref[...], preferred_element_type=F32)
    zz = asr + adr
    w = jnp.exp(jnp.where(zz >= 0.0, zz, 0.2 * zz))
    atd_ref[...] = adr
    ones1 = jnp.ones((NB, 1), F32)
    z14 = jnp.zeros((NB, 14), F32)
    z1 = jnp.zeros((NB, 1), F32)
    a1 = asr[:, 0:1]
    w1c = w[:, 0:1]
    htab_ref[0] = jnp.concatenate([h2[:, :C2], a1, ones1, z14], axis=1)
    htab_ref[1] = jnp.concatenate([h2[:, C2:], a1, ones1, z14], axis=1)
    minit_ref[0] = jnp.concatenate([h2[:, :C2] * w1c, z1, w1c, z14], axis=1)
    minit_ref[1] = jnp.concatenate([h2[:, C2:] * w1c, z1, w1c, z14], axis=1)

  return pl.pallas_call(
      body,
      grid=(GRID,),
      in_specs=[
          pl.BlockSpec((2, NB, CW1), lambda i: (0, i, 0)),
          pl.BlockSpec((HEADS * HID, OUT_CH), lambda i: (0, 0)),
          pl.BlockSpec((OUT_CH, 16), lambda i: (0, 0)),
          pl.BlockSpec((OUT_CH, 16), lambda i: (0, 0)),
          pl.BlockSpec((1, HEADS * HID), lambda i: (0, 0)),
          pl.BlockSpec((CW1, C1), lambda i: (0, 0)),
          pl.BlockSpec((CW1, C1), lambda i: (0, 0)),
      ],
      out_specs=[
          pl.BlockSpec((2, NB, CW2), lambda i: (0, i, 0)),
          pl.BlockSpec((NB, 16), lambda i: (i, 0)),
          pl.BlockSpec((2, NB, CW2), lambda i: (0, i, 0)),
      ],
      out_shape=[
          jax.ShapeDtypeStruct((2, N, CW2), F32),
          jax.ShapeDtypeStruct((N, 16), F32),
          jax.ShapeDtypeStruct((2, N, CW2), F32),
      ],
  )(macc1, W2, a2sp, a2dp, bias1, D0, D1)


def _stage_c(macc2, bias2):
  C2 = OUT_CH // 2
  CW2 = C2 + 16

  def body(m_ref, b2_ref, out_ref):
    o0 = m_ref[0][:, :C2] / (m_ref[0][:, C2 + 1:C2 + 2] + 1e-16)
    o1 = m_ref[1][:, :C2] / (m_ref[1][:, C2 + 1:C2 + 2] + 1e-16)
    o = jnp.concatenate([o0, o1], axis=1) + b2_ref[...]
    m = jnp.max(o, axis=1, keepdims=True)
    s = o - m
    out_ref[...] = s - jnp.log(jnp.sum(jnp.exp(s), axis=1, keepdims=True))

  return pl.pallas_call(
      body,
      grid=(GRID,),
      in_specs=[
          pl.BlockSpec((2, NB, CW2), lambda i: (0, i, 0)),
          pl.BlockSpec((1, OUT_CH), lambda i: (0, 0)),
      ],
      out_specs=pl.BlockSpec((NB, OUT_CH), lambda i: (i, 0)),
      out_shape=jax.ShapeDtypeStruct((N, OUT_CH), F32),
  )(macc2, bias2)


def _pad_rows(a, rows):
  return jnp.concatenate(
      [a, jnp.zeros((rows - a.shape[0],) + a.shape[1:], a.dtype)], axis=0)


@jax.jit
def kernel(x, edge_index, W1, att_src1, att_dst1, bias1,
           W2, att_src2, att_dst2, bias2):
  src = edge_index[0].astype(I32).reshape(NTILES, E // NTILES)
  dst = edge_index[1].astype(I32).reshape(NTILES, E // NTILES)
  padn = EPT - E // NTILES
  srcp = jnp.concatenate([src, jnp.zeros((NTILES, padn), I32)], axis=1)
  dstp = jnp.concatenate([dst, jnp.full((NTILES, padn), N, I32)], axis=1)
  srcp2 = jnp.stack([srcp, srcp + N])
  K1, K2 = 128, 64
  srcp2_1 = srcp2.reshape(2, NTILES, EPT // K1, K1)
  dstp_1 = dstp.reshape(NTILES, EPT // K1, K1)
  srcp2_2 = srcp2.reshape(2, NTILES, EPT // K2, K2)
  dstp_2 = dstp.reshape(NTILES, EPT // K2, K2)

  eye8 = jnp.eye(HEADS, dtype=F32)
  As1 = (att_src1[:, :, None] * eye8[:, None, :]).reshape(HEADS * HID, HEADS)
  Ad1 = (att_dst1[:, :, None] * eye8[:, None, :]).reshape(HEADS * HID, HEADS)
  zpad = jnp.zeros((HEADS * HID, 16 - HEADS), F32)
  As1p = jnp.concatenate([As1, zpad], axis=1)
  Ad1p = jnp.concatenate([Ad1, zpad], axis=1)

  g = jnp.arange(16)[:, None]
  ch = jnp.arange(4 * HID)[None, :]
  R0 = (g == ch // HID).astype(F32)
  R1 = (g == 4 + ch // HID).astype(F32)
  g48 = jnp.arange(48)[:, None]
  D0 = (g48 == 40 + ch // HID).astype(F32)
  D1 = (g48 == 44 + ch // HID).astype(F32)

  a2sp = jnp.concatenate(
      [att_src2.reshape(OUT_CH, 1), jnp.zeros((OUT_CH, 15), F32)], axis=1)
  a2dp = jnp.concatenate(
      [att_dst2.reshape(OUT_CH, 1), jnp.zeros((OUT_CH, 15), F32)], axis=1)

  htab1, atd1, minit1 = _stage_a(x, W1, As1p, Ad1p, R0, R1)
  macc1 = _sc_edge_pass(
      4 * HID, HEADS, K1, srcp2_1, dstp_1, _pad_rows(atd1, NPAD),
      htab1.reshape(NCORES * N, 4 * HID + 16),
      jnp.concatenate([_pad_rows(minit1[0], NPAD),
                       _pad_rows(minit1[1], NPAD)], axis=0))
  macc1 = macc1.reshape(NCORES, N, 4 * HID + 16)

  htab2, atd2, minit2 = _stage_b(
      macc1, W2, a2sp, a2dp, bias1.reshape(1, HEADS * HID), D0, D1)
  macc2 = _sc_edge_pass(
      OUT_CH // 2, 1, K2, srcp2_2, dstp_2, _pad_rows(atd2, NPAD),
      htab2.reshape(NCORES * N, OUT_CH // 2 + 16),
      jnp.concatenate([_pad_rows(minit2[0], NPAD),
                       _pad_rows(minit2[1], NPAD)], axis=0))
  macc2 = macc2.reshape(NCORES, N, OUT_CH // 2 + 16)

  return _stage_c(macc2, bias2.reshape(1, OUT_CH))


# speedup vs baseline: 34.8220x; 1.0687x over previous
import functools

import jax
import jax.numpy as jnp
from jax import lax
from jax.experimental import pallas as pl
from jax.experimental.pallas import tpu as pltpu
from jax.experimental.pallas import tpu_sc as plsc

N = 10000
E = 160000
IN_CH = 256
HID = 8
HEADS = 8
OUT_CH = 256

NPAD = 10008
NTILES = 16
NCORES = 2
EPT = 10240
INIT_RPT = 624
DRAIN = 624

F32 = jnp.float32
I32 = jnp.int32


def _sc_edge_pass(C, n_heads, K, srcp2, dstp, atd, htab, minit):
  CW = C + 16
  NCHUNK = EPT // K
  mesh = plsc.VectorSubcoreMesh(core_axis_name="c", subcore_axis_name="s")
  out_type = jax.ShapeDtypeStruct((NCORES * N, CW), F32)
  scratch = [
      pltpu.VMEM_SHARED((NPAD, CW), F32),
      [pltpu.VMEM((K,), I32) for _ in range(4)],
      [pltpu.VMEM((K,), I32) for _ in range(4)],
      [pltpu.VMEM((K, 16), F32) for _ in range(2)],
      [pltpu.VMEM((K, CW), F32) for _ in range(3)],
      [pltpu.SemaphoreType.DMA for _ in range(4)],
      [pltpu.SemaphoreType.DMA for _ in range(3)],
      [pltpu.SemaphoreType.DMA for _ in range(3)],
  ]

  @functools.partial(pl.kernel, out_type=out_type, mesh=mesh,
                     scratch_types=scratch,
                     compiler_params=pltpu.CompilerParams(
                         needs_layout_passes=False,
                         use_tc_tiling_on_sc=False))
  def k(src_h, dst_h, atd_h, htab_h, minit_h, mout_h,
        macc, sidx, didx, bv, hv, isem, gsem, ssem):
    cid = lax.axis_index("c")
    sid = lax.axis_index("s")
    lanes = lax.iota(I32, 16)

    r0 = sid * INIT_RPT
    pltpu.sync_copy(minit_h.at[pl.ds(cid * NPAD + r0, INIT_RPT), :],
                    macc.at[pl.ds(r0, INIT_RPT), :])

    @pl.when(sid == NTILES - 1)
    def _init_tail():
      t0 = NTILES * INIT_RPT
      pltpu.sync_copy(minit_h.at[pl.ds(cid * NPAD + t0, NPAD - t0), :],
                      macc.at[pl.ds(t0, NPAD - t0), :])

    plsc.subcore_barrier()

    def issue_i(j, s4):
      pltpu.async_copy(src_h.at[cid, sid, j], sidx[s4], isem[s4])
      pltpu.async_copy(dst_h.at[sid, j], didx[s4], isem[s4])

    def wait_i(j, s4):
      pltpu.make_async_copy(src_h.at[cid, sid, j], sidx[s4],
                            isem[s4]).wait()
      pltpu.make_async_copy(dst_h.at[sid, j], didx[s4], isem[s4]).wait()

    def issue_g(s4, b2, p):
      pltpu.async_copy(atd_h.at[didx[s4]], bv[b2], gsem[p])
      pltpu.async_copy(htab_h.at[sidx[s4]], hv[p], gsem[p])

    def wait_g(s4, b2, p):
      pltpu.make_async_copy(atd_h.at[didx[s4]], bv[b2], gsem[p]).wait()
      pltpu.make_async_copy(htab_h.at[sidx[s4]], hv[p], gsem[p]).wait()

    def issue_s(s4, p):
      pltpu.async_copy(hv[p], macc.at[didx[s4]], ssem[p], add=True)

    def wait_s(s4, p):
      pltpu.make_async_copy(hv[p], macc.at[didx[s4]], ssem[p]).wait()

    def compute(b2, p):
      hvp = hv[p]
      bvp = bv[b2]

      def edge(i):
        z = hvp[i, pl.ds(C, 16)] + bvp[i]
        z = jnp.where(z >= 0.0, z, 0.2 * z)
        w = jnp.exp(z)
        if n_heads == 1:
          wb = jnp.take_along_axis(w, lanes * 0, axis=0,
                                   mode="promise_in_bounds")
          for v in range(CW // 16):
            hvp[i, pl.ds(16 * v, 16)] = hvp[i, pl.ds(16 * v, 16)] * wb
        else:
          for v in range(C // 16):
            col = (lanes >> 3) + (4 * cid + 2 * v)
            wb = jnp.take_along_axis(w, col, axis=0,
                                     mode="promise_in_bounds")
            hvp[i, pl.ds(16 * v, 16)] = hvp[i, pl.ds(16 * v, 16)] * wb
          wb = jnp.take_along_axis(w, lanes & 7, axis=0,
                                   mode="promise_in_bounds")
          hvp[i, pl.ds(C, 16)] = hvp[i, pl.ds(C, 16)] * wb

      def body2(t, c2):
        edge(2 * t)
        edge(2 * t + 1)
        return c2
      lax.fori_loop(0, K // 2, body2, 0)

    def step(j, s4, s4n, s4n2, b2, b2n, p, q, first):
      wait_g(s4, b2, p)
      if not first:
        wait_s(s4n2, q)
      wait_i(j + 1, s4n)
      issue_g(s4n, b2n, q)
      issue_i(j + 2, s4n2)
      compute(b2, p)
      issue_s(s4, p)

    issue_i(0, 0)
    issue_i(1, 1)
    wait_i(0, 0)
    issue_g(0, 0, 0)
    step(0, 0, 1, 2, 0, 1, 0, 1, True)
    step(1, 1, 2, 3, 1, 0, 1, 2, True)

    M = (NCHUNK - 4) // 12

    def pipe12(t, c2):
      for b in range(12):
        j = 2 + 12 * t + b
        step(j, (2 + b) % 4, (3 + b) % 4, b % 4, b % 2, (1 + b) % 2,
             (2 + b) % 3, b % 3, False)
      return c2
    lax.fori_loop(0, M, pipe12, 0)

    for j in range(2 + 12 * M, NCHUNK):
      p = j % 3
      q = (j + 1) % 3
      wait_g(j % 4, j % 2, p)
      wait_s((j + 2) % 4, q)
      if j + 1 < NCHUNK:
        wait_i(j + 1, (j + 1) % 4)
        issue_g((j + 1) % 4, (j + 1) % 2, q)
      if j + 2 < NCHUNK:
        issue_i(j + 2, (j + 2) % 4)
      compute(j % 2, p)
      issue_s(j % 4, p)
    wait_s((NCHUNK - 2) % 4, (NCHUNK - 2) % 3)
    wait_s((NCHUNK - 1) % 4, (NCHUNK - 1) % 3)
    plsc.subcore_barrier()

    q0 = sid * DRAIN
    pltpu.sync_copy(macc.at[pl.ds(q0, DRAIN), :],
                    mout_h.at[pl.ds(cid * N + q0, DRAIN), :])

    @pl.when(sid == NTILES - 1)
    def _drain_tail():
      t0 = NTILES * DRAIN
      pltpu.sync_copy(macc.at[pl.ds(t0, N - t0), :],
                      mout_h.at[pl.ds(cid * N + t0, N - t0), :])

  return k(srcp2, dstp, atd, htab, minit)


NB = 1000
GRID = N // NB


def _stage_a(x, W1, As1p, Ad1p, R0, R1):
  C = 4 * HID
  CW = C + 16

  def body(x_ref, w1_ref, as_ref, ad_ref, r0_ref, r1_ref,
           htab_ref, atd_ref, minit_ref):
    h = jnp.dot(x_ref[...], w1_ref[...], preferred_element_type=F32)
    asr = jnp.dot(h, as_ref[...], preferred_element_type=F32)
    adr = jnp.dot(h, ad_ref[...], preferred_element_type=F32)
    z = asr + adr
    w = jnp.exp(jnp.where(z >= 0.0, z, 0.2 * z))
    atd_ref[...] = adr
    ones8 = jnp.ones((NB, 8), F32)
    zeros8 = jnp.zeros((NB, 8), F32)
    htab_ref[0] = jnp.concatenate([h[:, :C], asr[:, :8], ones8], axis=1)
    htab_ref[1] = jnp.concatenate([h[:, C:], asr[:, :8], ones8], axis=1)
    w0 = jnp.dot(w, r0_ref[...], preferred_element_type=F32)
    w1 = jnp.dot(w, r1_ref[...], preferred_element_type=F32)
    minit_ref[0] = jnp.concatenate([w0 * h[:, :C], zeros8, w[:, :8]], axis=1)
    minit_ref[1] = jnp.concatenate([w1 * h[:, C:], zeros8, w[:, :8]], axis=1)

  return pl.pallas_call(
      body,
      grid=(GRID,),
      in_specs=[
          pl.BlockSpec((NB, IN_CH), lambda i: (i, 0)),
          pl.BlockSpec((IN_CH, HEADS * HID), lambda i: (0, 0)),
          pl.BlockSpec((HEADS * HID, 16), lambda i: (0, 0)),
          pl.BlockSpec((HEADS * HID, 16), lambda i: (0, 0)),
          pl.BlockSpec((16, C), lambda i: (0, 0)),
          pl.BlockSpec((16, C), lambda i: (0, 0)),
      ],
      out_specs=[
          pl.BlockSpec((2, NB, CW), lambda i: (0, i, 0)),
          pl.BlockSpec((NB, 16), lambda i: (i, 0)),
          pl.BlockSpec((2, NB, CW), lambda i: (0, i, 0)),
      ],
      out_shape=[
          jax.ShapeDtypeStruct((2, N, CW), F32),
          jax.ShapeDtypeStruct((NPAD, 16), F32),
          jax.ShapeDtypeStruct((2, NPAD, CW), F32),
      ],
  )(x, W1, As1p, Ad1p, R0, R1)


def _stage_b(macc1, W2, a2sp, a2dp, bias1, D0, D1):
  C1 = 4 * HID
  CW1 = C1 + 16
  C2 = OUT_CH // 2
  CW2 = C2 + 16

  def body(m_ref, w2_ref, a2s_ref, a2d_ref, b1_ref, d0_ref, d1_ref,
           htab_ref, atd_ref, minit_ref):
    o0 = m_ref[0][:, :C1] / (jnp.dot(m_ref[0], d0_ref[...],
                                     preferred_element_type=F32) + 1e-16)
    o1 = m_ref[1][:, :C1] / (jnp.dot(m_ref[1], d1_ref[...],
                                     preferred_element_type=F32) + 1e-16)
    z = jnp.concatenate([o0, o1], axis=1) + b1_ref[...]
    z = jnp.where(z > 0.0, z, jnp.exp(z) - 1.0)
    h2 = jnp.dot(z, w2_ref[...], preferred_element_type=F32)
    asr = jnp.dot(h2, a2s_ref[...], preferred_element_type=F32)
    adr = jnp.dot(h2, a2d_ref[...], preferred_element_type=F32)
    zz = asr + adr
    w = jnp.exp(jnp.where(zz >= 0.0, zz, 0.2 * zz))
    atd_ref[...] = adr
    ones1 = jnp.ones((NB, 1), F32)
    z14 = jnp.zeros((NB, 14), F32)
    z1 = jnp.zeros((NB, 1), F32)
    a1 = asr[:, 0:1]
    w1c = w[:, 0:1]
    htab_ref[0] = jnp.concatenate([h2[:, :C2], a1, ones1, z14], axis=1)
    htab_ref[1] = jnp.concatenate([h2[:, C2:], a1, ones1, z14], axis=1)
    minit_ref[0] = jnp.concatenate([h2[:, :C2] * w1c, z1, w1c, z14], axis=1)
    minit_ref[1] = jnp.concatenate([h2[:, C2:] * w1c, z1, w1c, z14], axis=1)

  return pl.pallas_call(
      body,
      grid=(GRID,),
      in_specs=[
          pl.BlockSpec((2, NB, CW1), lambda i: (0, i, 0)),
          pl.BlockSpec((HEADS * HID, OUT_CH), lambda i: (0, 0)),
          pl.BlockSpec((OUT_CH, 16), lambda i: (0, 0)),
          pl.BlockSpec((OUT_CH, 16), lambda i: (0, 0)),
          pl.BlockSpec((1, HEADS * HID), lambda i: (0, 0)),
          pl.BlockSpec((CW1, C1), lambda i: (0, 0)),
          pl.BlockSpec((CW1, C1), lambda i: (0, 0)),
      ],
      out_specs=[
          pl.BlockSpec((2, NB, CW2), lambda i: (0, i, 0)),
          pl.BlockSpec((NB, 16), lambda i: (i, 0)),
          pl.BlockSpec((2, NB, CW2), lambda i: (0, i, 0)),
      ],
      out_shape=[
          jax.ShapeDtypeStruct((2, N, CW2), F32),
          jax.ShapeDtypeStruct((NPAD, 16), F32),
          jax.ShapeDtypeStruct((2, NPAD, CW2), F32),
      ],
  )(macc1, W2, a2sp, a2dp, bias1, D0, D1)


def _stage_c(macc2, bias2):
  C2 = OUT_CH // 2
  CW2 = C2 + 16

  def body(m_ref, b2_ref, out_ref):
    o0 = m_ref[0][:, :C2] / (m_ref[0][:, C2 + 1:C2 + 2] + 1e-16)
    o1 = m_ref[1][:, :C2] / (m_ref[1][:, C2 + 1:C2 + 2] + 1e-16)
    o = jnp.concatenate([o0, o1], axis=1) + b2_ref[...]
    m = jnp.max(o, axis=1, keepdims=True)
    s = o - m
    out_ref[...] = s - jnp.log(jnp.sum(jnp.exp(s), axis=1, keepdims=True))

  return pl.pallas_call(
      body,
      grid=(GRID,),
      in_specs=[
          pl.BlockSpec((2, NB, CW2), lambda i: (0, i, 0)),
          pl.BlockSpec((1, OUT_CH), lambda i: (0, 0)),
      ],
      out_specs=pl.BlockSpec((NB, OUT_CH), lambda i: (i, 0)),
      out_shape=jax.ShapeDtypeStruct((N, OUT_CH), F32),
  )(macc2, bias2)


def _pad_rows(a, rows):
  return jnp.concatenate(
      [a, jnp.zeros((rows - a.shape[0],) + a.shape[1:], a.dtype)], axis=0)


@jax.jit
def kernel(x, edge_index, W1, att_src1, att_dst1, bias1,
           W2, att_src2, att_dst2, bias2):
  src = edge_index[0].astype(I32).reshape(NTILES, E // NTILES)
  dst = edge_index[1].astype(I32).reshape(NTILES, E // NTILES)
  padn = EPT - E // NTILES
  srcp = jnp.concatenate([src, jnp.zeros((NTILES, padn), I32)], axis=1)
  dstp = jnp.concatenate([dst, jnp.full((NTILES, padn), N, I32)], axis=1)
  srcp2 = jnp.stack([srcp, srcp + N])
  K1, K2 = 128, 64
  srcp2_1 = srcp2.reshape(2, NTILES, EPT // K1, K1)
  dstp_1 = dstp.reshape(NTILES, EPT // K1, K1)
  srcp2_2 = srcp2.reshape(2, NTILES, EPT // K2, K2)
  dstp_2 = dstp.reshape(NTILES, EPT // K2, K2)

  eye8 = jnp.eye(HEADS, dtype=F32)
  As1 = (att_src1[:, :, None] * eye8[:, None, :]).reshape(HEADS * HID, HEADS)
  Ad1 = (att_dst1[:, :, None] * eye8[:, None, :]).reshape(HEADS * HID, HEADS)
  zpad = jnp.zeros((HEADS * HID, 16 - HEADS), F32)
  As1p = jnp.concatenate([As1, zpad], axis=1)
  Ad1p = jnp.concatenate([Ad1, zpad], axis=1)

  g = jnp.arange(16)[:, None]
  ch = jnp.arange(4 * HID)[None, :]
  R0 = (g == ch // HID).astype(F32)
  R1 = (g == 4 + ch // HID).astype(F32)
  g48 = jnp.arange(48)[:, None]
  D0 = (g48 == 40 + ch // HID).astype(F32)
  D1 = (g48 == 44 + ch // HID).astype(F32)

  a2sp = jnp.concatenate(
      [att_src2.reshape(OUT_CH, 1), jnp.zeros((OUT_CH, 15), F32)], axis=1)
  a2dp = jnp.concatenate(
      [att_dst2.reshape(OUT_CH, 1), jnp.zeros((OUT_CH, 15), F32)], axis=1)

  htab1, atd1, minit1 = _stage_a(x, W1, As1p, Ad1p, R0, R1)
  macc1 = _sc_edge_pass(
      4 * HID, HEADS, K1, srcp2_1, dstp_1, atd1,
      htab1.reshape(NCORES * N, 4 * HID + 16),
      minit1.reshape(NCORES * NPAD, 4 * HID + 16))
  macc1 = macc1.reshape(NCORES, N, 4 * HID + 16)

  htab2, atd2, minit2 = _stage_b(
      macc1, W2, a2sp, a2dp, bias1.reshape(1, HEADS * HID), D0, D1)
  macc2 = _sc_edge_pass(
      OUT_CH // 2, 1, K2, srcp2_2, dstp_2, atd2,
      htab2.reshape(NCORES * N, OUT_CH // 2 + 16),
      minit2.reshape(NCORES * NPAD, OUT_CH // 2 + 16))
  macc2 = macc2.reshape(NCORES, N, OUT_CH // 2 + 16)

  return _stage_c(macc2, bias2.reshape(1, OUT_CH))


# speedup vs baseline: 36.9886x; 1.0622x over previous
import functools

import jax
import jax.numpy as jnp
from jax import lax
from jax.experimental import pallas as pl
from jax.experimental.pallas import tpu as pltpu
from jax.experimental.pallas import tpu_sc as plsc

N = 10000
E = 160000
IN_CH = 256
HID = 8
HEADS = 8
OUT_CH = 256

NPAD = 10008
NTILES = 16
NCORES = 2
EPT = 10240
INIT_RPT = 624
DRAIN = 624

F32 = jnp.float32
I32 = jnp.int32


def _sc_edge_pass(C, n_heads, K, ept, edge_split, srcp2, dstp, atd, htab,
                  minit):
  CW = C + 16
  NCHUNK = ept // K
  mesh = plsc.VectorSubcoreMesh(core_axis_name="c", subcore_axis_name="s")
  out_type = jax.ShapeDtypeStruct((NCORES * N, CW), F32)
  scratch = [
      pltpu.VMEM_SHARED((NPAD, CW), F32),
      [pltpu.VMEM((K,), I32) for _ in range(4)],
      [pltpu.VMEM((K,), I32) for _ in range(4)],
      [pltpu.VMEM((K, 16), F32) for _ in range(2)],
      [pltpu.VMEM((K, CW), F32) for _ in range(3)],
      [pltpu.SemaphoreType.DMA for _ in range(4)],
      [pltpu.SemaphoreType.DMA for _ in range(3)],
      [pltpu.SemaphoreType.DMA for _ in range(3)],
  ]

  @functools.partial(pl.kernel, out_type=out_type, mesh=mesh,
                     scratch_types=scratch,
                     compiler_params=pltpu.CompilerParams(
                         needs_layout_passes=False,
                         use_tc_tiling_on_sc=False))
  def k(src_h, dst_h, atd_h, htab_h, minit_h, mout_h,
        macc, sidx, didx, bv, hv, isem, gsem, ssem):
    cid = lax.axis_index("c")
    sid = lax.axis_index("s")
    lanes = lax.iota(I32, 16)

    r0 = sid * INIT_RPT
    pltpu.sync_copy(minit_h.at[pl.ds(cid * NPAD + r0, INIT_RPT), :],
                    macc.at[pl.ds(r0, INIT_RPT), :])

    @pl.when(sid == NTILES - 1)
    def _init_tail():
      t0 = NTILES * INIT_RPT
      pltpu.sync_copy(minit_h.at[pl.ds(cid * NPAD + t0, NPAD - t0), :],
                      macc.at[pl.ds(t0, NPAD - t0), :])

    plsc.subcore_barrier()

    def _dst_ref(j):
      return dst_h.at[cid, sid, j] if edge_split else dst_h.at[sid, j]

    def issue_i(j, s4):
      pltpu.async_copy(src_h.at[cid, sid, j], sidx[s4], isem[s4])
      pltpu.async_copy(_dst_ref(j), didx[s4], isem[s4])

    def wait_i(j, s4):
      pltpu.make_async_copy(src_h.at[cid, sid, j], sidx[s4],
                            isem[s4]).wait()
      pltpu.make_async_copy(_dst_ref(j), didx[s4], isem[s4]).wait()

    def issue_g(s4, b2, p):
      pltpu.async_copy(atd_h.at[didx[s4]], bv[b2], gsem[p])
      pltpu.async_copy(htab_h.at[sidx[s4]], hv[p], gsem[p])

    def wait_g(s4, b2, p):
      pltpu.make_async_copy(atd_h.at[didx[s4]], bv[b2], gsem[p]).wait()
      pltpu.make_async_copy(htab_h.at[sidx[s4]], hv[p], gsem[p]).wait()

    def issue_s(s4, p):
      pltpu.async_copy(hv[p], macc.at[didx[s4]], ssem[p], add=True)

    def wait_s(s4, p):
      pltpu.make_async_copy(hv[p], macc.at[didx[s4]], ssem[p]).wait()

    def compute(b2, p):
      hvp = hv[p]
      bvp = bv[b2]

      def edge(i):
        z = hvp[i, pl.ds(C, 16)] + bvp[i]
        z = jnp.where(z >= 0.0, z, 0.2 * z)
        w = jnp.exp(z)
        if n_heads == 1:
          wb = jnp.take_along_axis(w, lanes * 0, axis=0,
                                   mode="promise_in_bounds")
          for v in range(CW // 16):
            hvp[i, pl.ds(16 * v, 16)] = hvp[i, pl.ds(16 * v, 16)] * wb
        else:
          for v in range(C // 16):
            col = (lanes >> 3) + 2 * v
            wb = jnp.take_along_axis(w, col, axis=0,
                                     mode="promise_in_bounds")
            hvp[i, pl.ds(16 * v, 16)] = hvp[i, pl.ds(16 * v, 16)] * wb
          wb = jnp.take_along_axis(w, lanes & 7, axis=0,
                                   mode="promise_in_bounds")
          hvp[i, pl.ds(C, 16)] = hvp[i, pl.ds(C, 16)] * wb

      def body2(t, c2):
        edge(2 * t)
        edge(2 * t + 1)
        return c2
      lax.fori_loop(0, K // 2, body2, 0)

    def step(j, s4, s4n, s4n2, b2, b2n, p, q, first):
      wait_g(s4, b2, p)
      if not first:
        wait_s(s4n2, q)
      wait_i(j + 1, s4n)
      issue_g(s4n, b2n, q)
      issue_i(j + 2, s4n2)
      compute(b2, p)
      issue_s(s4, p)

    issue_i(0, 0)
    issue_i(1, 1)
    wait_i(0, 0)
    issue_g(0, 0, 0)
    step(0, 0, 1, 2, 0, 1, 0, 1, True)
    step(1, 1, 2, 3, 1, 0, 1, 2, True)

    M = (NCHUNK - 4) // 12

    def pipe12(t, c2):
      for b in range(12):
        j = 2 + 12 * t + b
        step(j, (2 + b) % 4, (3 + b) % 4, b % 4, b % 2, (1 + b) % 2,
             (2 + b) % 3, b % 3, False)
      return c2
    lax.fori_loop(0, M, pipe12, 0)

    for j in range(2 + 12 * M, NCHUNK):
      p = j % 3
      q = (j + 1) % 3
      wait_g(j % 4, j % 2, p)
      wait_s((j + 2) % 4, q)
      if j + 1 < NCHUNK:
        wait_i(j + 1, (j + 1) % 4)
        issue_g((j + 1) % 4, (j + 1) % 2, q)
      if j + 2 < NCHUNK:
        issue_i(j + 2, (j + 2) % 4)
      compute(j % 2, p)
      issue_s(j % 4, p)
    wait_s((NCHUNK - 2) % 4, (NCHUNK - 2) % 3)
    wait_s((NCHUNK - 1) % 4, (NCHUNK - 1) % 3)
    plsc.subcore_barrier()

    q0 = sid * DRAIN
    pltpu.sync_copy(macc.at[pl.ds(q0, DRAIN), :],
                    mout_h.at[pl.ds(cid * N + q0, DRAIN), :])

    @pl.when(sid == NTILES - 1)
    def _drain_tail():
      t0 = NTILES * DRAIN
      pltpu.sync_copy(macc.at[pl.ds(t0, N - t0), :],
                      mout_h.at[pl.ds(cid * N + t0, N - t0), :])

  return k(srcp2, dstp, atd, htab, minit)


NB = 1000
GRID = N // NB


def _stage_a(x, W1, As1p, Ad1p, R64):
  C = HEADS * HID
  CW = C + 16

  def body(x_ref, w1_ref, as_ref, ad_ref, r_ref,
           htab_ref, atd_ref, minit_ref):
    h = jnp.dot(x_ref[...], w1_ref[...], preferred_element_type=F32)
    asr = jnp.dot(h, as_ref[...], preferred_element_type=F32)
    adr = jnp.dot(h, ad_ref[...], preferred_element_type=F32)
    z = asr + adr
    w = jnp.exp(jnp.where(z >= 0.0, z, 0.2 * z))
    atd_ref[...] = adr
    ones8 = jnp.ones((NB, 8), F32)
    zeros8 = jnp.zeros((NB, 8), F32)
    htab_ref[...] = jnp.concatenate([h, asr[:, :8], ones8], axis=1)
    wrep = jnp.dot(w, r_ref[...], preferred_element_type=F32)
    minit_ref[0] = jnp.concatenate([wrep * h, zeros8, w[:, :8]], axis=1)
    minit_ref[1] = jnp.zeros((NB, CW), F32)

  return pl.pallas_call(
      body,
      grid=(GRID,),
      in_specs=[
          pl.BlockSpec((NB, IN_CH), lambda i: (i, 0)),
          pl.BlockSpec((IN_CH, HEADS * HID), lambda i: (0, 0)),
          pl.BlockSpec((HEADS * HID, 16), lambda i: (0, 0)),
          pl.BlockSpec((HEADS * HID, 16), lambda i: (0, 0)),
          pl.BlockSpec((16, C), lambda i: (0, 0)),
      ],
      out_specs=[
          pl.BlockSpec((NB, CW), lambda i: (i, 0)),
          pl.BlockSpec((NB, 16), lambda i: (i, 0)),
          pl.BlockSpec((2, NB, CW), lambda i: (0, i, 0)),
      ],
      out_shape=[
          jax.ShapeDtypeStruct((N, CW), F32),
          jax.ShapeDtypeStruct((NPAD, 16), F32),
          jax.ShapeDtypeStruct((2, NPAD, CW), F32),
      ],
  )(x, W1, As1p, Ad1p, R64)


def _stage_b(macc1, W2, a2sp, a2dp, bias1, D64):
  C1 = HEADS * HID
  CW1 = C1 + 16
  C2 = OUT_CH // 2
  CW2 = C2 + 16

  def body(m_ref, w2_ref, a2s_ref, a2d_ref, b1_ref, d_ref,
           htab_ref, atd_ref, minit_ref):
    m = m_ref[0] + m_ref[1]
    o = m[:, :C1] / (jnp.dot(m, d_ref[...],
                             preferred_element_type=F32) + 1e-16)
    z = o + b1_ref[...]
    z = jnp.where(z > 0.0, z, jnp.exp(z) - 1.0)
    h2 = jnp.dot(z, w2_ref[...], preferred_element_type=F32)
    asr = jnp.dot(h2, a2s_ref[...], preferred_element_type=F32)
    adr = jnp.dot(h2, a2d_ref[...], preferred_element_type=F32)
    zz = asr + adr
    w = jnp.exp(jnp.where(zz >= 0.0, zz, 0.2 * zz))
    atd_ref[...] = adr
    ones1 = jnp.ones((NB, 1), F32)
    z14 = jnp.zeros((NB, 14), F32)
    z1 = jnp.zeros((NB, 1), F32)
    a1 = asr[:, 0:1]
    w1c = w[:, 0:1]
    htab_ref[0] = jnp.concatenate([h2[:, :C2], a1, ones1, z14], axis=1)
    htab_ref[1] = jnp.concatenate([h2[:, C2:], a1, ones1, z14], axis=1)
    minit_ref[0] = jnp.concatenate([h2[:, :C2] * w1c, z1, w1c, z14], axis=1)
    minit_ref[1] = jnp.concatenate([h2[:, C2:] * w1c, z1, w1c, z14], axis=1)

  return pl.pallas_call(
      body,
      grid=(GRID,),
      in_specs=[
          pl.BlockSpec((2, NB, CW1), lambda i: (0, i, 0)),
          pl.BlockSpec((HEADS * HID, OUT_CH), lambda i: (0, 0)),
          pl.BlockSpec((OUT_CH, 16), lambda i: (0, 0)),
          pl.BlockSpec((OUT_CH, 16), lambda i: (0, 0)),
          pl.BlockSpec((1, HEADS * HID), lambda i: (0, 0)),
          pl.BlockSpec((CW1, C1), lambda i: (0, 0)),
      ],
      out_specs=[
          pl.BlockSpec((2, NB, CW2), lambda i: (0, i, 0)),
          pl.BlockSpec((NB, 16), lambda i: (i, 0)),
          pl.BlockSpec((2, NB, CW2), lambda i: (0, i, 0)),
      ],
      out_shape=[
          jax.ShapeDtypeStruct((2, N, CW2), F32),
          jax.ShapeDtypeStruct((NPAD, 16), F32),
          jax.ShapeDtypeStruct((2, NPAD, CW2), F32),
      ],
  )(macc1, W2, a2sp, a2dp, bias1, D64)


def _stage_c(macc2, bias2):
  C2 = OUT_CH // 2
  CW2 = C2 + 16

  def body(m_ref, b2_ref, out_ref):
    o0 = m_ref[0][:, :C2] / (m_ref[0][:, C2 + 1:C2 + 2] + 1e-16)
    o1 = m_ref[1][:, :C2] / (m_ref[1][:, C2 + 1:C2 + 2] + 1e-16)
    o = jnp.concatenate([o0, o1], axis=1) + b2_ref[...]
    m = jnp.max(o, axis=1, keepdims=True)
    s = o - m
    out_ref[...] = s - jnp.log(jnp.sum(jnp.exp(s), axis=1, keepdims=True))

  return pl.pallas_call(
      body,
      grid=(GRID,),
      in_specs=[
          pl.BlockSpec((2, NB, CW2), lambda i: (0, i, 0)),
          pl.BlockSpec((1, OUT_CH), lambda i: (0, 0)),
      ],
      out_specs=pl.BlockSpec((NB, OUT_CH), lambda i: (i, 0)),
      out_shape=jax.ShapeDtypeStruct((N, OUT_CH), F32),
  )(macc2, bias2)


def _pad_rows(a, rows):
  return jnp.concatenate(
      [a, jnp.zeros((rows - a.shape[0],) + a.shape[1:], a.dtype)], axis=0)


@jax.jit
def kernel(x, edge_index, W1, att_src1, att_dst1, bias1,
           W2, att_src2, att_dst2, bias2):
  K1, K2 = 128, 64
  EPT1 = 5120
  src32 = edge_index[0].astype(I32).reshape(NCORES, NTILES, E // 32)
  dst32 = edge_index[1].astype(I32).reshape(NCORES, NTILES, E // 32)
  pad1 = EPT1 - E // 32
  srcp_1 = jnp.concatenate(
      [src32, jnp.zeros((NCORES, NTILES, pad1), I32)],
      axis=2).reshape(NCORES, NTILES, EPT1 // K1, K1)
  dstp_1 = jnp.concatenate(
      [dst32, jnp.full((NCORES, NTILES, pad1), N, I32)],
      axis=2).reshape(NCORES, NTILES, EPT1 // K1, K1)
  src16 = edge_index[0].astype(I32).reshape(NTILES, E // NTILES)
  dst16 = edge_index[1].astype(I32).reshape(NTILES, E // NTILES)
  padn = EPT - E // NTILES
  srcp = jnp.concatenate([src16, jnp.zeros((NTILES, padn), I32)], axis=1)
  dstp = jnp.concatenate([dst16, jnp.full((NTILES, padn), N, I32)], axis=1)
  srcp2_2 = jnp.stack([srcp, srcp + N]).reshape(2, NTILES, EPT // K2, K2)
  dstp_2 = dstp.reshape(NTILES, EPT // K2, K2)

  eye8 = jnp.eye(HEADS, dtype=F32)
  As1 = (att_src1[:, :, None] * eye8[:, None, :]).reshape(HEADS * HID, HEADS)
  Ad1 = (att_dst1[:, :, None] * eye8[:, None, :]).reshape(HEADS * HID, HEADS)
  zpad = jnp.zeros((HEADS * HID, 16 - HEADS), F32)
  As1p = jnp.concatenate([As1, zpad], axis=1)
  Ad1p = jnp.concatenate([Ad1, zpad], axis=1)

  g = jnp.arange(16)[:, None]
  ch = jnp.arange(HEADS * HID)[None, :]
  R64 = (g == ch // HID).astype(F32)
  g80 = jnp.arange(80)[:, None]
  D64 = (g80 == 72 + ch // HID).astype(F32)

  a2sp = jnp.concatenate(
      [att_src2.reshape(OUT_CH, 1), jnp.zeros((OUT_CH, 15), F32)], axis=1)
  a2dp = jnp.concatenate(
      [att_dst2.reshape(OUT_CH, 1), jnp.zeros((OUT_CH, 15), F32)], axis=1)

  htab1, atd1, minit1 = _stage_a(x, W1, As1p, Ad1p, R64)
  macc1 = _sc_edge_pass(
      HEADS * HID, HEADS, K1, EPT1, True, srcp_1, dstp_1, atd1,
      htab1, minit1.reshape(NCORES * NPAD, HEADS * HID + 16))
  macc1 = macc1.reshape(NCORES, N, HEADS * HID + 16)

  htab2, atd2, minit2 = _stage_b(
      macc1, W2, a2sp, a2dp, bias1.reshape(1, HEADS * HID), D64)
  macc2 = _sc_edge_pass(
      OUT_CH // 2, 1, K2, EPT, False, srcp2_2, dstp_2, atd2,
      htab2.reshape(NCORES * N, OUT_CH // 2 + 16),
      minit2.reshape(NCORES * NPAD, OUT_CH // 2 + 16))
  macc2 = macc2.reshape(NCORES, N, OUT_CH // 2 + 16)

  return _stage_c(macc2, bias2.reshape(1, OUT_CH))
